# Initial kernel scaffold; baseline (speedup 1.0000x reference)
import jax
import jax.numpy as jnp
from jax.experimental import pallas as pl


def kernel(x, edge_index, edge_attr, state, batch, W1, b1, g1, be1, W2, b2, g2, be2, W3, b3, g3, be3):
    raise NotImplementedError("write your pallas kernel here")


# speedup vs baseline: 4.6662x; 4.6662x over previous
import functools

import jax
import jax.numpy as jnp
from jax import lax
from jax.experimental import pallas as pl
from jax.experimental.pallas import tpu as pltpu
from jax.experimental.pallas import tpu_sc as plsc

N = 100000
E = 1600000
D = 32
G = 512

ECA = 800
ECHUNKS_A = (E // 16) // ECA
NCA = 800
NCHUNKS_A = N // NCA

ECB = 2000
ECHUNKS_B = (E // 16) // ECB
NCB = 2000
NCHUNKS_B = N // NCB
GC = 1000
GCHUNKS = N // GC


def _sc_sums_body(idx_hbm, edge_hbm, sums_hbm, idx_v, rows_v, sums_sp):
    c = lax.axis_index("c")
    s = lax.axis_index("s")

    def fill(i, _):
        rows_v[i] = jnp.zeros((16,), jnp.float32)
        return 0
    lax.fori_loop(0, ECA, fill, 0)

    def zero_chunk(k):
        pltpu.sync_copy(rows_v, sums_sp.at[pl.ds(k * NCA, NCA)])

    for j in range(7):
        zero_chunk(s + 16 * j)

    @pl.when(s < NCHUNKS_A - 112)
    def _():
        zero_chunk(s + 112)

    plsc.subcore_barrier()

    ebase = s * (E // 16)

    def edge_step(k, _):
        e0 = ebase + k * ECA
        pltpu.sync_copy(idx_hbm.at[pl.ds(e0, ECA)], idx_v)
        pltpu.sync_copy(edge_hbm.at[pl.ds(e0, ECA), pl.ds(c * 16, 16)],
                        rows_v)
        pltpu.sync_copy(rows_v, sums_sp.at[idx_v], add=True)
        return 0

    lax.fori_loop(0, ECHUNKS_A, edge_step, 0)

    plsc.subcore_barrier()

    def wb_chunk(k):
        pltpu.sync_copy(sums_sp.at[pl.ds(k * NCA, NCA)],
                        sums_hbm.at[pl.ds(k * NCA, NCA), pl.ds(c * 16, 16)])

    for j in range(7):
        wb_chunk(s + 16 * j)

    @pl.when(s < NCHUNKS_A - 112)
    def _():
        wb_chunk(s + 112)


def _sc_aux_body(idx_hbm, state_hbm, batch_hbm, cntp_hbm, sg_hbm,
                 zero_v, ones_v, cidx_v, gidx_v, grow_v, gsem, cnt_sp):
    c = lax.axis_index("c")
    s = lax.axis_index("s")

    def fill(i, _):
        zero_v[pl.ds(i * 16, 16)] = jnp.zeros((16,), jnp.float32)
        ones_v[pl.ds(i * 16, 16)] = jnp.full((16,), 1.0, jnp.float32)
        return 0
    lax.fori_loop(0, ECB // 16, fill, 0)

    def zero_chunk(k):
        pltpu.sync_copy(zero_v, cnt_sp.at[pl.ds(k * NCB, NCB)])

    for j in range(3):
        zero_chunk(s + 16 * j)

    @pl.when(s < NCHUNKS_B - 48)
    def _():
        zero_chunk(s + 48)

    plsc.subcore_barrier()

    ebase = s * (E // 16)

    def cnt_step(k, _):
        @pl.when((k % 2) == c)
        def _():
            e0 = ebase + k * ECB
            pltpu.sync_copy(idx_hbm.at[pl.ds(e0, ECB)], cidx_v)
            pltpu.sync_copy(ones_v, cnt_sp.at[cidx_v], add=True)
        return 0

    lax.fori_loop(0, ECHUNKS_B, cnt_step, 0)

    w = s * 2 + c

    def gather_chunk(k):
        pltpu.sync_copy(batch_hbm.at[pl.ds(k * GC, GC)], gidx_v)
        pltpu.async_copy(state_hbm.at[gidx_v], grow_v, gsem).wait()
        pltpu.sync_copy(grow_v, sg_hbm.at[pl.ds(k * GC, GC)])

    for j in range(3):
        gather_chunk(w + 32 * j)

    @pl.when(w < GCHUNKS - 96)
    def _():
        gather_chunk(w + 96)

    plsc.subcore_barrier()

    def wb_chunk(k):
        pltpu.sync_copy(cnt_sp.at[pl.ds(k * NCB, NCB)],
                        cntp_hbm.at[c, pl.ds(k * NCB, NCB)])

    for j in range(3):
        wb_chunk(s + 16 * j)

    @pl.when(s < NCHUNKS_B - 48)
    def _():
        wb_chunk(s + 48)


def _make_sc_kernels():
    mesh = plsc.VectorSubcoreMesh(core_axis_name="c", subcore_axis_name="s")
    params = pltpu.CompilerParams(use_tc_tiling_on_sc=False)
    sums_k = pl.kernel(
        _sc_sums_body,
        out_type=jax.ShapeDtypeStruct((N, D), jnp.float32),
        mesh=mesh,
        compiler_params=params,
        scratch_types=[
            pltpu.VMEM((ECA,), jnp.int32),
            pltpu.VMEM((ECA, 16), jnp.float32),
            pltpu.VMEM_SHARED((N, 16), jnp.float32),
        ],
    )
    aux_k = pl.kernel(
        _sc_aux_body,
        out_type=(
            jax.ShapeDtypeStruct((2, N), jnp.float32),
            jax.ShapeDtypeStruct((N, D), jnp.float32),
        ),
        mesh=mesh,
        compiler_params=params,
        scratch_types=[
            pltpu.VMEM((NCB,), jnp.float32),
            pltpu.VMEM((ECB,), jnp.float32),
            pltpu.VMEM((ECB,), jnp.int32),
            pltpu.VMEM((GC,), jnp.int32),
            pltpu.VMEM((GC, D), jnp.float32),
            pltpu.SemaphoreType.DMA,
            pltpu.VMEM_SHARED((N,), jnp.float32),
        ],
    )
    return sums_k, aux_k


BT = 2000
NBLK = N // BT


def _p1_body(x_ref, sums_ref, c0_ref, c1_ref, sg_ref,
             w1x_ref, w1v_ref, w1s_ref, b1_ref,
             r1_ref, s1_ref, q1_ref):
    i = pl.program_id(0)
    ct = c0_ref[...] + c1_ref[...]
    recip = 1.0 / jnp.maximum(ct, 1.0)
    h = jnp.dot(x_ref[...], w1x_ref[...], preferred_element_type=jnp.float32)
    h = h + jnp.dot(sums_ref[...], w1v_ref[...],
                    preferred_element_type=jnp.float32) * recip
    h = h + jnp.dot(sg_ref[...], w1s_ref[...],
                    preferred_element_type=jnp.float32)
    h = h + b1_ref[...]
    r = jnp.maximum(h, 0.0)
    r1_ref[...] = r
    ps = jnp.sum(r, axis=0, keepdims=True)
    pq = jnp.sum(r * r, axis=0, keepdims=True)

    @pl.when(i == 0)
    def _():
        s1_ref[...] = ps
        q1_ref[...] = pq

    @pl.when(i > 0)
    def _():
        s1_ref[...] += ps
        q1_ref[...] += pq


def _p2_body(r1_ref, s1_ref, q1_ref, w2_ref, b2_ref, g1_ref, be1_ref,
             r2_ref, s2_ref, q2_ref, *, relu):
    i = pl.program_id(0)
    mu = s1_ref[...] * (1.0 / N)
    var = q1_ref[...] * (1.0 / N) - mu * mu
    a = g1_ref[...] * lax.rsqrt(var + 1e-5)
    sh = be1_ref[...] - mu * a
    bn = r1_ref[...] * a + sh
    z = jnp.dot(bn, w2_ref[...], preferred_element_type=jnp.float32)
    z = z + b2_ref[...]
    if relu:
        z = jnp.maximum(z, 0.0)
    r2_ref[...] = z
    ps = jnp.sum(z, axis=0, keepdims=True)
    pq = jnp.sum(z * z, axis=0, keepdims=True)

    @pl.when(i == 0)
    def _():
        s2_ref[...] = ps
        q2_ref[...] = pq

    @pl.when(i > 0)
    def _():
        s2_ref[...] += ps
        q2_ref[...] += pq


def _p4_body(h3_ref, s3_ref, q3_ref, g3_ref, be3_ref, out_ref):
    mu = s3_ref[...] * (1.0 / N)
    var = q3_ref[...] * (1.0 / N) - mu * mu
    a = g3_ref[...] * lax.rsqrt(var + 1e-5)
    sh = be3_ref[...] - mu * a
    out_ref[...] = h3_ref[...] * a + sh


def _row_spec():
    return pl.BlockSpec((BT, D), lambda i: (i, 0))


def _cnt_spec():
    return pl.BlockSpec((BT, 1), lambda i: (i, 0))


def _const_spec(shape):
    return pl.BlockSpec(shape, lambda i: (0, 0))


def _stats_shape():
    return jax.ShapeDtypeStruct((1, D), jnp.float32)


@jax.jit
def kernel(x, edge_index, edge_attr, state, batch,
           W1, b1, g1, be1, W2, b2, g2, be2, W3, b3, g3, be3):
    idx = edge_index[0, :]

    sums_k, aux_k = _make_sc_kernels()
    sums = sums_k(idx, edge_attr)
    cntp, sg = aux_k(idx, state, batch)

    c0 = cntp[0].reshape(N, 1)
    c1 = cntp[1].reshape(N, 1)

    w1x = W1[:, :D].T
    w1v = W1[:, D:2 * D].T
    w1s = W1[:, 2 * D:].T
    b1r = b1.reshape(1, D)

    r1, s1, q1 = pl.pallas_call(
        _p1_body,
        grid=(NBLK,),
        in_specs=[
            _row_spec(), _row_spec(), _cnt_spec(), _cnt_spec(), _row_spec(),
            _const_spec((D, D)), _const_spec((D, D)), _const_spec((D, D)),
            _const_spec((1, D)),
        ],
        out_specs=[_row_spec(), _const_spec((1, D)), _const_spec((1, D))],
        out_shape=[
            jax.ShapeDtypeStruct((N, D), jnp.float32),
            _stats_shape(), _stats_shape(),
        ],
    )(x, sums, c0, c1, sg, w1x, w1v, w1s, b1r)

    def mid_pass(r, s_, q_, w, b, g, be, relu):
        return pl.pallas_call(
            functools.partial(_p2_body, relu=relu),
            grid=(NBLK,),
            in_specs=[
                _row_spec(),
                _const_spec((1, D)), _const_spec((1, D)),
                _const_spec((D, D)), _const_spec((1, D)),
                _const_spec((1, D)), _const_spec((1, D)),
            ],
            out_specs=[_row_spec(), _const_spec((1, D)), _const_spec((1, D))],
            out_shape=[
                jax.ShapeDtypeStruct((N, D), jnp.float32),
                _stats_shape(), _stats_shape(),
            ],
        )(r, s_, q_, w.T, b.reshape(1, D), g.reshape(1, D), be.reshape(1, D))

    r2, s2, q2 = mid_pass(r1, s1, q1, W2, b2, g1, be1, relu=True)
    h3, s3, q3 = mid_pass(r2, s2, q2, W3, b3, g2, be2, relu=False)

    out = pl.pallas_call(
        _p4_body,
        grid=(NBLK,),
        in_specs=[
            _row_spec(),
            _const_spec((1, D)), _const_spec((1, D)),
            _const_spec((1, D)), _const_spec((1, D)),
        ],
        out_specs=_row_spec(),
        out_shape=jax.ShapeDtypeStruct((N, D), jnp.float32),
    )(h3, s3, q3, g3.reshape(1, D), be3.reshape(1, D))

    return out


# speedup vs baseline: 4.7066x; 1.0087x over previous
import functools

import jax
import jax.numpy as jnp
from jax import lax
from jax.experimental import pallas as pl
from jax.experimental.pallas import tpu as pltpu
from jax.experimental.pallas import tpu_sc as plsc

N = 100000
E = 1600000
D = 32
G = 512

ECA = 800
ECHUNKS_A = (E // 16) // ECA
NCA = 800
NCHUNKS_A = N // NCA

GC = 1000
GCHUNKS = N // GC


def _sc_mean_body(eidx_hbm, edge_hbm, vmean_hbm,
                  idx_v, rows_v, ones_v, cntb_v, sums_sp, cnt_sp):
    c = lax.axis_index("c")
    s = lax.axis_index("s")

    def fill_rows(i, _):
        rows_v[i] = jnp.zeros((16,), jnp.float32)
        return 0
    lax.fori_loop(0, ECA, fill_rows, 0)

    def fill_ones(i, _):
        ones_v[pl.ds(i * 16, 16)] = jnp.full((16,), 1.0, jnp.float32)
        cntb_v[pl.ds(i * 16, 16)] = jnp.zeros((16,), jnp.float32)
        return 0
    lax.fori_loop(0, ECA // 16, fill_ones, 0)

    def zero_chunk(k):
        pltpu.sync_copy(rows_v, sums_sp.at[pl.ds(k * NCA, NCA)])
        pltpu.sync_copy(cntb_v, cnt_sp.at[pl.ds(k * NCA, NCA)])

    for j in range(7):
        zero_chunk(s + 16 * j)

    @pl.when(s < NCHUNKS_A - 112)
    def _():
        zero_chunk(s + 112)

    plsc.subcore_barrier()

    ebase = s * (E // 16)

    def edge_step(k, _):
        e0 = ebase + k * ECA
        pltpu.sync_copy(eidx_hbm.at[0, pl.ds(e0, ECA)], idx_v)
        pltpu.sync_copy(edge_hbm.at[pl.ds(e0, ECA), pl.ds(c * 16, 16)],
                        rows_v)
        pltpu.sync_copy(rows_v, sums_sp.at[idx_v], add=True)
        pltpu.sync_copy(ones_v, cnt_sp.at[idx_v], add=True)
        return 0

    lax.fori_loop(0, ECHUNKS_A, edge_step, 0)

    plsc.subcore_barrier()

    def norm_chunk(k):
        pltpu.sync_copy(sums_sp.at[pl.ds(k * NCA, NCA)], rows_v)
        pltpu.sync_copy(cnt_sp.at[pl.ds(k * NCA, NCA)], cntb_v)

        def group_step(g, _):
            base = g * 16
            cv = cntb_v[pl.ds(base, 16)]
            rv = 1.0 / jnp.maximum(cv, 1.0)
            for i in range(16):
                rows_v[base + i] = rows_v[base + i] * rv[i]
            return 0

        lax.fori_loop(0, NCA // 16, group_step, 0)
        pltpu.sync_copy(rows_v,
                        vmean_hbm.at[pl.ds(k * NCA, NCA), pl.ds(c * 16, 16)])

    for j in range(7):
        norm_chunk(s + 16 * j)

    @pl.when(s < NCHUNKS_A - 112)
    def _():
        norm_chunk(s + 112)


def _sc_gather_body(state_hbm, batch_hbm, sg_hbm, gidx_v, grow_v, gsem):
    c = lax.axis_index("c")
    s = lax.axis_index("s")
    w = s * 2 + c

    def gather_chunk(k):
        pltpu.sync_copy(batch_hbm.at[pl.ds(k * GC, GC)], gidx_v)
        pltpu.async_copy(state_hbm.at[gidx_v], grow_v, gsem).wait()
        pltpu.sync_copy(grow_v, sg_hbm.at[pl.ds(k * GC, GC)])

    for j in range(3):
        gather_chunk(w + 32 * j)

    @pl.when(w < GCHUNKS - 96)
    def _():
        gather_chunk(w + 96)


def _make_sc_kernels():
    mesh = plsc.VectorSubcoreMesh(core_axis_name="c", subcore_axis_name="s")
    params = pltpu.CompilerParams(use_tc_tiling_on_sc=False)
    mean_k = pl.kernel(
        _sc_mean_body,
        out_type=jax.ShapeDtypeStruct((N, D), jnp.float32),
        mesh=mesh,
        compiler_params=params,
        scratch_types=[
            pltpu.VMEM((ECA,), jnp.int32),
            pltpu.VMEM((ECA, 16), jnp.float32),
            pltpu.VMEM((ECA,), jnp.float32),
            pltpu.VMEM((ECA,), jnp.float32),
            pltpu.VMEM_SHARED((N, 16), jnp.float32),
            pltpu.VMEM_SHARED((N,), jnp.float32),
        ],
    )
    gather_k = pl.kernel(
        _sc_gather_body,
        out_type=jax.ShapeDtypeStruct((N, D), jnp.float32),
        mesh=mesh,
        compiler_params=params,
        scratch_types=[
            pltpu.VMEM((GC,), jnp.int32),
            pltpu.VMEM((GC, D), jnp.float32),
            pltpu.SemaphoreType.DMA,
        ],
    )
    return mean_k, gather_k


BT = 2000
NBLK = N // BT


def _p1_body(x_ref, vm_ref, sg_ref, w1x_ref, w1v_ref, w1s_ref, b1_ref,
             r1_ref, s1_ref, q1_ref):
    i = pl.program_id(0)
    h = jnp.dot(x_ref[...], w1x_ref[...], preferred_element_type=jnp.float32)
    h = h + jnp.dot(vm_ref[...], w1v_ref[...],
                    preferred_element_type=jnp.float32)
    h = h + jnp.dot(sg_ref[...], w1s_ref[...],
                    preferred_element_type=jnp.float32)
    h = h + b1_ref[...]
    r = jnp.maximum(h, 0.0)
    r1_ref[...] = r
    ps = jnp.sum(r, axis=0, keepdims=True)
    pq = jnp.sum(r * r, axis=0, keepdims=True)

    @pl.when(i == 0)
    def _():
        s1_ref[...] = ps
        q1_ref[...] = pq

    @pl.when(i > 0)
    def _():
        s1_ref[...] += ps
        q1_ref[...] += pq


def _p2_body(r1_ref, s1_ref, q1_ref, w2_ref, b2_ref, g1_ref, be1_ref,
             r2_ref, s2_ref, q2_ref, *, relu):
    i = pl.program_id(0)
    mu = s1_ref[...] * (1.0 / N)
    var = q1_ref[...] * (1.0 / N) - mu * mu
    a = g1_ref[...] * lax.rsqrt(var + 1e-5)
    sh = be1_ref[...] - mu * a
    bn = r1_ref[...] * a + sh
    z = jnp.dot(bn, w2_ref[...], preferred_element_type=jnp.float32)
    z = z + b2_ref[...]
    if relu:
        z = jnp.maximum(z, 0.0)
    r2_ref[...] = z
    ps = jnp.sum(z, axis=0, keepdims=True)
    pq = jnp.sum(z * z, axis=0, keepdims=True)

    @pl.when(i == 0)
    def _():
        s2_ref[...] = ps
        q2_ref[...] = pq

    @pl.when(i > 0)
    def _():
        s2_ref[...] += ps
        q2_ref[...] += pq


def _p4_body(h3_ref, s3_ref, q3_ref, g3_ref, be3_ref, out_ref):
    mu = s3_ref[...] * (1.0 / N)
    var = q3_ref[...] * (1.0 / N) - mu * mu
    a = g3_ref[...] * lax.rsqrt(var + 1e-5)
    sh = be3_ref[...] - mu * a
    out_ref[...] = h3_ref[...] * a + sh


def _row_spec():
    return pl.BlockSpec((BT, D), lambda i: (i, 0))


def _const_spec(shape):
    return pl.BlockSpec(shape, lambda i: (0, 0))


def _stats_shape():
    return jax.ShapeDtypeStruct((1, D), jnp.float32)


@jax.jit
def kernel(x, edge_index, edge_attr, state, batch,
           W1, b1, g1, be1, W2, b2, g2, be2, W3, b3, g3, be3):
    mean_k, gather_k = _make_sc_kernels()
    vmean = mean_k(edge_index, edge_attr)
    sg = gather_k(state, batch)

    w1x = W1[:, :D].T
    w1v = W1[:, D:2 * D].T
    w1s = W1[:, 2 * D:].T
    b1r = b1.reshape(1, D)

    r1, s1, q1 = pl.pallas_call(
        _p1_body,
        grid=(NBLK,),
        in_specs=[
            _row_spec(), _row_spec(), _row_spec(),
            _const_spec((D, D)), _const_spec((D, D)), _const_spec((D, D)),
            _const_spec((1, D)),
        ],
        out_specs=[_row_spec(), _const_spec((1, D)), _const_spec((1, D))],
        out_shape=[
            jax.ShapeDtypeStruct((N, D), jnp.float32),
            _stats_shape(), _stats_shape(),
        ],
    )(x, vmean, sg, w1x, w1v, w1s, b1r)

    def mid_pass(r, s_, q_, w, b, g, be, relu):
        return pl.pallas_call(
            functools.partial(_p2_body, relu=relu),
            grid=(NBLK,),
            in_specs=[
                _row_spec(),
                _const_spec((1, D)), _const_spec((1, D)),
                _const_spec((D, D)), _const_spec((1, D)),
                _const_spec((1, D)), _const_spec((1, D)),
            ],
            out_specs=[_row_spec(), _const_spec((1, D)), _const_spec((1, D))],
            out_shape=[
                jax.ShapeDtypeStruct((N, D), jnp.float32),
                _stats_shape(), _stats_shape(),
            ],
        )(r, s_, q_, w.T, b.reshape(1, D), g.reshape(1, D), be.reshape(1, D))

    r2, s2, q2 = mid_pass(r1, s1, q1, W2, b2, g1, be1, relu=True)
    h3, s3, q3 = mid_pass(r2, s2, q2, W3, b3, g2, be2, relu=False)

    out = pl.pallas_call(
        _p4_body,
        grid=(NBLK,),
        in_specs=[
            _row_spec(),
            _const_spec((1, D)), _const_spec((1, D)),
            _const_spec((1, D)), _const_spec((1, D)),
        ],
        out_specs=_row_spec(),
        out_shape=jax.ShapeDtypeStruct((N, D), jnp.float32),
    )(h3, s3, q3, g3.reshape(1, D), be3.reshape(1, D))

    return out


# speedup vs baseline: 5.2519x; 1.1158x over previous
import functools

import jax
import jax.numpy as jnp
from jax import lax
from jax.experimental import pallas as pl
from jax.experimental.pallas import tpu as pltpu
from jax.experimental.pallas import tpu_sc as plsc

N = 100000
E = 1600000
D = 32
G = 512

ECA = 800
ECHUNKS_A = (E // 16) // ECA
NZ = 800
NZCHUNKS = N // NZ
NC = 400
NCHUNKS = N // NC

GC = 800
GCHUNKS = N // GC


def _sc_mean_body(eidx_hbm, edge_hbm, vmt_hbm,
                  idx_v, rows_v, ones_v, trows_v, sums_sp, cnt_sp):
    c = lax.axis_index("c")
    s = lax.axis_index("s")

    def fill_rows(i, _):
        rows_v[i] = jnp.zeros((16,), jnp.float32)
        return 0
    lax.fori_loop(0, ECA, fill_rows, 0)

    def fill_zero1d(i, _):
        ones_v[pl.ds(i * 16, 16)] = jnp.zeros((16,), jnp.float32)
        return 0
    lax.fori_loop(0, ECA // 16, fill_zero1d, 0)

    def zero_chunk(k):
        pltpu.sync_copy(rows_v, sums_sp.at[pl.ds(k * NZ, NZ)])
        pltpu.sync_copy(ones_v, cnt_sp.at[pl.ds(k * NZ, NZ)])

    for j in range(7):
        zero_chunk(s + 16 * j)

    @pl.when(s < NZCHUNKS - 112)
    def _():
        zero_chunk(s + 112)

    def fill_ones(i, _):
        ones_v[pl.ds(i * 16, 16)] = jnp.full((16,), 1.0, jnp.float32)
        return 0
    lax.fori_loop(0, ECA // 16, fill_ones, 0)

    plsc.subcore_barrier()

    ebase = s * (E // 16)

    def edge_step(k, _):
        e0 = ebase + k * ECA
        pltpu.sync_copy(eidx_hbm.at[0, pl.ds(e0, ECA)], idx_v)
        pltpu.sync_copy(edge_hbm.at[pl.ds(e0, ECA), pl.ds(c * 16, 16)],
                        rows_v)
        pltpu.sync_copy(rows_v, sums_sp.at[idx_v], add=True)
        pltpu.sync_copy(ones_v, cnt_sp.at[idx_v], add=True)
        return 0

    lax.fori_loop(0, ECHUNKS_A, edge_step, 0)

    plsc.subcore_barrier()

    lanes = lax.iota(jnp.int32, 16)

    def norm_chunk(k):
        pltpu.sync_copy(sums_sp.at[pl.ds(k * NC, NC)],
                        rows_v.at[pl.ds(0, NC)])
        pltpu.sync_copy(cnt_sp.at[pl.ds(k * NC, NC)], trows_v.at[0])

        def group_step(g, _):
            base = g * 16
            cv = trows_v[0, pl.ds(base, 16)]
            rv = 1.0 / jnp.maximum(cv, 1.0)
            ridx = base + lanes
            for f in range(16):
                col = plsc.load_gather(
                    rows_v, [ridx, jnp.full((16,), f, jnp.int32)])
                trows_v[f, pl.ds(base, 16)] = col * rv
            return 0

        lax.fori_loop(0, NC // 16, group_step, 0)
        pltpu.sync_copy(trows_v,
                        vmt_hbm.at[pl.ds(c * 16, 16), pl.ds(k * NC, NC)])

    for j in range(15):
        norm_chunk(s + 16 * j)

    @pl.when(s < NCHUNKS - 240)
    def _():
        norm_chunk(s + 240)


def _sc_gather_body(state_hbm, batch_hbm, sgt_hbm,
                    gidx_v, grow_v, tgrow_v, gsem):
    c = lax.axis_index("c")
    s = lax.axis_index("s")
    w = s * 2 + c
    lanes = lax.iota(jnp.int32, 16)

    def gather_chunk(k):
        pltpu.sync_copy(batch_hbm.at[pl.ds(k * GC, GC)], gidx_v)
        pltpu.async_copy(state_hbm.at[gidx_v], grow_v, gsem).wait()

        def group_step(g, _):
            base = g * 16
            ridx = base + lanes
            for f in range(D):
                col = plsc.load_gather(
                    grow_v, [ridx, jnp.full((16,), f, jnp.int32)])
                tgrow_v[f, pl.ds(base, 16)] = col
            return 0

        lax.fori_loop(0, GC // 16, group_step, 0)
        pltpu.sync_copy(tgrow_v, sgt_hbm.at[:, pl.ds(k * GC, GC)])

    for j in range(3):
        gather_chunk(w + 32 * j)

    @pl.when(w < GCHUNKS - 96)
    def _():
        gather_chunk(w + 96)


def _make_sc_kernels():
    mesh = plsc.VectorSubcoreMesh(core_axis_name="c", subcore_axis_name="s")
    params = pltpu.CompilerParams(use_tc_tiling_on_sc=False,
                                  needs_layout_passes=False)
    mean_k = pl.kernel(
        _sc_mean_body,
        out_type=jax.ShapeDtypeStruct((D, N), jnp.float32),
        mesh=mesh,
        compiler_params=params,
        scratch_types=[
            pltpu.VMEM((ECA,), jnp.int32),
            pltpu.VMEM((ECA, 16), jnp.float32),
            pltpu.VMEM((ECA,), jnp.float32),
            pltpu.VMEM((16, NC), jnp.float32),
            pltpu.VMEM_SHARED((N, 16), jnp.float32),
            pltpu.VMEM_SHARED((N,), jnp.float32),
        ],
    )
    gather_k = pl.kernel(
        _sc_gather_body,
        out_type=jax.ShapeDtypeStruct((D, N), jnp.float32),
        mesh=mesh,
        compiler_params=params,
        scratch_types=[
            pltpu.VMEM((GC,), jnp.int32),
            pltpu.VMEM((GC, D), jnp.float32),
            pltpu.VMEM((D, GC), jnp.float32),
            pltpu.SemaphoreType.DMA,
        ],
    )
    return mean_k, gather_k


BT = 4096
NBLK = -(-N // BT)


def _valid_mask(i):
    cols = i * BT + lax.broadcasted_iota(jnp.int32, (D, BT), 1)
    return cols < N


def _p1_body(x_ref, vm_ref, sg_ref, w1x_ref, w1v_ref, w1s_ref, b1_ref,
             r1_ref, s1_ref, q1_ref):
    i = pl.program_id(0)
    h = jnp.dot(w1x_ref[...], x_ref[...], preferred_element_type=jnp.float32)
    h = h + jnp.dot(w1v_ref[...], vm_ref[...],
                    preferred_element_type=jnp.float32)
    h = h + jnp.dot(w1s_ref[...], sg_ref[...],
                    preferred_element_type=jnp.float32)
    h = h + b1_ref[...]
    r = jnp.maximum(h, 0.0)
    r1_ref[...] = r
    rm = jnp.where(_valid_mask(i), r, 0.0)
    ps = jnp.sum(rm, axis=1, keepdims=True)
    pq = jnp.sum(rm * rm, axis=1, keepdims=True)

    @pl.when(i == 0)
    def _():
        s1_ref[...] = ps
        q1_ref[...] = pq

    @pl.when(i > 0)
    def _():
        s1_ref[...] += ps
        q1_ref[...] += pq


def _p2_body(r1_ref, s1_ref, q1_ref, w2_ref, b2_ref, g1_ref, be1_ref,
             r2_ref, s2_ref, q2_ref, *, relu):
    i = pl.program_id(0)
    mu = s1_ref[...] * (1.0 / N)
    var = q1_ref[...] * (1.0 / N) - mu * mu
    a = g1_ref[...] * lax.rsqrt(var + 1e-5)
    sh = be1_ref[...] - mu * a
    bn = r1_ref[...] * a + sh
    z = jnp.dot(w2_ref[...], bn, preferred_element_type=jnp.float32)
    z = z + b2_ref[...]
    if relu:
        z = jnp.maximum(z, 0.0)
    r2_ref[...] = z
    zm = jnp.where(_valid_mask(i), z, 0.0)
    ps = jnp.sum(zm, axis=1, keepdims=True)
    pq = jnp.sum(zm * zm, axis=1, keepdims=True)

    @pl.when(i == 0)
    def _():
        s2_ref[...] = ps
        q2_ref[...] = pq

    @pl.when(i > 0)
    def _():
        s2_ref[...] += ps
        q2_ref[...] += pq


def _p4_body(h3_ref, s3_ref, q3_ref, g3_ref, be3_ref, out_ref):
    mu = s3_ref[...] * (1.0 / N)
    var = q3_ref[...] * (1.0 / N) - mu * mu
    a = g3_ref[...] * lax.rsqrt(var + 1e-5)
    sh = be3_ref[...] - mu * a
    out_ref[...] = h3_ref[...] * a + sh


def _col_spec():
    return pl.BlockSpec((D, BT), lambda i: (0, i))


def _const_spec(shape):
    return pl.BlockSpec(shape, lambda i: (0, 0))


def _stats_shape():
    return jax.ShapeDtypeStruct((D, 1), jnp.float32)


@jax.jit
def kernel(x, edge_index, edge_attr, state, batch,
           W1, b1, g1, be1, W2, b2, g2, be2, W3, b3, g3, be3):
    mean_k, gather_k = _make_sc_kernels()
    vmt = mean_k(edge_index, edge_attr)
    sgt = gather_k(state, batch)

    xt = x.T
    w1x = W1[:, :D]
    w1v = W1[:, D:2 * D]
    w1s = W1[:, 2 * D:]

    r1, s1, q1 = pl.pallas_call(
        _p1_body,
        grid=(NBLK,),
        in_specs=[
            _col_spec(), _col_spec(), _col_spec(),
            _const_spec((D, D)), _const_spec((D, D)), _const_spec((D, D)),
            _const_spec((D, 1)),
        ],
        out_specs=[_col_spec(), _const_spec((D, 1)), _const_spec((D, 1))],
        out_shape=[
            jax.ShapeDtypeStruct((D, N), jnp.float32),
            _stats_shape(), _stats_shape(),
        ],
    )(xt, vmt, sgt, w1x, w1v, w1s, b1.reshape(D, 1))

    def mid_pass(r, s_, q_, w, b, g, be, relu):
        return pl.pallas_call(
            functools.partial(_p2_body, relu=relu),
            grid=(NBLK,),
            in_specs=[
                _col_spec(),
                _const_spec((D, 1)), _const_spec((D, 1)),
                _const_spec((D, D)), _const_spec((D, 1)),
                _const_spec((D, 1)), _const_spec((D, 1)),
            ],
            out_specs=[_col_spec(), _const_spec((D, 1)), _const_spec((D, 1))],
            out_shape=[
                jax.ShapeDtypeStruct((D, N), jnp.float32),
                _stats_shape(), _stats_shape(),
            ],
        )(r, s_, q_, w, b.reshape(D, 1), g.reshape(D, 1), be.reshape(D, 1))

    r2, s2, q2 = mid_pass(r1, s1, q1, W2, b2, g1, be1, relu=True)
    h3, s3, q3 = mid_pass(r2, s2, q2, W3, b3, g2, be2, relu=False)

    outt = pl.pallas_call(
        _p4_body,
        grid=(NBLK,),
        in_specs=[
            _col_spec(),
            _const_spec((D, 1)), _const_spec((D, 1)),
            _const_spec((D, 1)), _const_spec((D, 1)),
        ],
        out_specs=_col_spec(),
        out_shape=jax.ShapeDtypeStruct((D, N), jnp.float32),
    )(h3, s3, q3, g3.reshape(D, 1), be3.reshape(D, 1))

    return outt.T
